# Initial kernel scaffold; baseline (speedup 1.0000x reference)
import jax
import jax.numpy as jnp
from jax.experimental import pallas as pl


def kernel(cls_pred, cls_target, loc_pred, loc_target, anchors):
    raise NotImplementedError("write your pallas kernel here")


# speedup vs baseline: 769.3178x; 769.3178x over previous
import jax
import jax.numpy as jnp
from jax.experimental import pallas as pl
from jax.experimental.pallas import tpu as pltpu

_R = 20000
_ROWS = 160
_LANES = 128
_RP = _ROWS * _LANES
_IOU_T = 0.7
_HALF_BATCH = 200
_SIGMA = 10.0


def _ohem_kernel(cls0_ref, cls1_ref, ct_ref, lp0_ref, lp1_ref, lt0_ref,
                 lt1_ref, ax1_ref, ay1_ref, ax2_ref, ay2_ref,
                 cls_out, loc_out):
    shape = (_ROWS, _LANES)
    lin = (jax.lax.broadcasted_iota(jnp.int32, shape, 0) * _LANES
           + jax.lax.broadcasted_iota(jnp.int32, shape, 1))
    c0 = cls0_ref[...]
    c1 = cls1_ref[...]
    t = ct_ref[...]
    zero = jnp.float32(0.0)

    mx = jnp.maximum(c0, c1)
    s0 = c0 - mx
    s1 = c1 - mx
    lse = jnp.log(jnp.exp(s0) + jnp.exp(s1))
    ce = lse - jnp.where(t == 1, s1, s0)

    def _sl1(d):
        less_one = (d < 1.0 / _SIGMA).astype(jnp.float32)
        return (less_one * 0.5 * d ** 2 * _SIGMA
                + jnp.abs(1 - less_one) * (d - 0.5 / _SIGMA))

    sl = (_sl1(jnp.abs(lt0_ref[...] - lp0_ref[...]))
          + _sl1(jnp.abs(lt1_ref[...] - lp1_ref[...])))

    x1 = ax1_ref[...]
    y1 = ay1_ref[...]
    x2 = ax2_ref[...]
    y2 = ay2_ref[...]
    areas = (x2 - x1) * (y2 - y1)

    pos_m = t == 1
    neg_m = t == 0
    total_pc = jnp.sum(jnp.where(pos_m, ce, zero))
    total_pl = jnp.sum(jnp.where(pos_m, sl, zero))
    total_nc = jnp.sum(jnp.where(neg_m, ce, zero))

    def run_nms(loss, init_alive, with_sl):
        def body(_, state):
            key, cnt, acc_c, acc_s = state
            m = jnp.max(key)
            has = m >= zero
            eqm = key == m
            i = jnp.min(jnp.where(eqm, lin, jnp.int32(2 ** 30)))
            pick = lin == i

            def ext(v):
                return jnp.sum(jnp.where(pick, v, zero))

            x1i = ext(x1)
            y1i = ext(y1)
            x2i = ext(x2)
            y2i = ext(y2)
            ce_i = ext(ce)
            area_i = (x2i - x1i) * (y2i - y1i)
            inter = (jnp.maximum(jnp.minimum(x2i, x2) - jnp.maximum(x1i, x1), zero)
                     * jnp.maximum(jnp.minimum(y2i, y2) - jnp.maximum(y1i, y1), zero))
            iou = inter / (area_i + areas - inter)
            kill = pick | jnp.logical_not(iou <= _IOU_T)
            key = jnp.where(kill & has, -1.0, key)
            take = has & (cnt < _HALF_BATCH)
            acc_c = acc_c + jnp.where(take, ce_i, zero)
            if with_sl:
                acc_s = acc_s + jnp.where(take, ext(sl), zero)
            cnt = cnt + jnp.where(has, 1, 0)
            return key, cnt, acc_c, acc_s

        init_key = jnp.where(init_alive, loss, -1.0)
        return jax.lax.fori_loop(
            0, _HALF_BATCH + 1, body,
            (init_key, jnp.int32(0), zero, zero))

    _, cnt_p, acc_pc, acc_pl = run_nms(ce + sl, pos_m, True)
    _, cnt_n, acc_nc, _ = run_nms(ce, neg_m, False)

    trunc_p = cnt_p > _HALF_BATCH
    trunc_n = cnt_n > _HALF_BATCH
    keep_p = jnp.minimum(cnt_p, _HALF_BATCH)
    keep_n = jnp.minimum(cnt_n, _HALF_BATCH)
    sum_pc = jnp.where(trunc_p, acc_pc, total_pc)
    sum_pl = jnp.where(trunc_p, acc_pl, total_pl)
    sum_nc = jnp.where(trunc_n, acc_nc, total_nc)
    cls_out[0, 0] = (sum_nc + sum_pc) / (keep_p + keep_n).astype(jnp.float32)
    loc_out[0, 0] = sum_pl / keep_p.astype(jnp.float32)


def kernel(cls_pred, cls_target, loc_pred, loc_target, anchors):
    cp = cls_pred[0]
    ct = cls_target[0, 0].astype(jnp.int32)
    lp = loc_pred[0]
    lt = loc_target[0]
    an = anchors[0]
    pad = _RP - _R

    def p2(x, v=0):
        return jnp.pad(x, (0, pad), constant_values=v).reshape(_ROWS, _LANES)

    args = (p2(cp[:, 0]), p2(cp[:, 1]), p2(ct, 2),
            p2(lp[:, 0]), p2(lp[:, 1]), p2(lt[:, 0]), p2(lt[:, 1]),
            p2(an[:, 0]), p2(an[:, 1]), p2(an[:, 2]), p2(an[:, 3]))
    cls_o, loc_o = pl.pallas_call(
        _ohem_kernel,
        out_shape=(jax.ShapeDtypeStruct((1, 1), jnp.float32),
                   jax.ShapeDtypeStruct((1, 1), jnp.float32)),
        out_specs=(pl.BlockSpec(memory_space=pltpu.SMEM),
                   pl.BlockSpec(memory_space=pltpu.SMEM)),
    )(*args)
    return cls_o[0, 0], loc_o[0, 0]


# speedup vs baseline: 938.0752x; 1.2194x over previous
import jax
import jax.numpy as jnp
from jax.experimental import pallas as pl
from jax.experimental.pallas import tpu as pltpu

_R = 20000
_ROWS = 160
_LANES = 128
_RP = _ROWS * _LANES
_IOU_T = 0.7
_HALF_BATCH = 200
_SIGMA = 10.0


def _ohem_kernel(cls0_ref, cls1_ref, ct_ref, lp0_ref, lp1_ref, lt0_ref,
                 lt1_ref, ax1_ref, ay1_ref, ax2_ref, ay2_ref,
                 cls_out, loc_out,
                 ce_ref, sl_ref, areas_ref, keyp_ref, keyn_ref):
    shape = (_ROWS, _LANES)
    lin = (jax.lax.broadcasted_iota(jnp.int32, shape, 0) * _LANES
           + jax.lax.broadcasted_iota(jnp.int32, shape, 1))
    lane_iota = jax.lax.broadcasted_iota(jnp.int32, (1, _LANES), 1)
    zero = jnp.float32(0.0)
    t = ct_ref[...]

    c0 = cls0_ref[...]
    c1 = cls1_ref[...]
    mx = jnp.maximum(c0, c1)
    s0 = c0 - mx
    s1 = c1 - mx
    lse = jnp.log(jnp.exp(s0) + jnp.exp(s1))
    ce = lse - jnp.where(t == 1, s1, s0)

    def _sl1(d):
        less_one = (d < 1.0 / _SIGMA).astype(jnp.float32)
        return (less_one * 0.5 * d ** 2 * _SIGMA
                + jnp.abs(1 - less_one) * (d - 0.5 / _SIGMA))

    sl = (_sl1(jnp.abs(lt0_ref[...] - lp0_ref[...]))
          + _sl1(jnp.abs(lt1_ref[...] - lp1_ref[...])))

    areas_ref[...] = ((ax2_ref[...] - ax1_ref[...])
                      * (ay2_ref[...] - ay1_ref[...]))
    ce_ref[...] = ce
    sl_ref[...] = sl

    pos_m = t == 1
    neg_m = t == 0
    total_pc = jnp.sum(jnp.where(pos_m, ce, zero))
    total_pl = jnp.sum(jnp.where(pos_m, sl, zero))
    total_nc = jnp.sum(jnp.where(neg_m, ce, zero))
    keyp_ref[...] = jnp.where(pos_m, ce + sl, -1.0)
    keyn_ref[...] = jnp.where(neg_m, ce, -1.0)

    def ext(ref, row, lmask):
        return jnp.sum(jnp.where(lmask, ref[pl.ds(row, 1), :], zero))

    def step(key_ref, with_sl, cnt, acc_c, acc_s):
        key = key_ref[...]
        m = jnp.max(key)
        has = m >= zero
        i = jnp.min(jnp.where(key == m, lin, jnp.int32(2 ** 30)))
        ic = jnp.minimum(i, jnp.int32(_RP - 1))
        row = jax.lax.shift_right_logical(ic, 7)
        lmask = lane_iota == jnp.bitwise_and(ic, 127)
        x1i = ext(ax1_ref, row, lmask)
        y1i = ext(ay1_ref, row, lmask)
        x2i = ext(ax2_ref, row, lmask)
        y2i = ext(ay2_ref, row, lmask)
        ce_i = ext(ce_ref, row, lmask)
        area_i = (x2i - x1i) * (y2i - y1i)
        inter = (jnp.maximum(jnp.minimum(x2i, ax2_ref[...])
                             - jnp.maximum(x1i, ax1_ref[...]), zero)
                 * jnp.maximum(jnp.minimum(y2i, ay2_ref[...])
                               - jnp.maximum(y1i, ay1_ref[...]), zero))
        iou = inter / ((area_i + areas_ref[...]) - inter)
        kill = (lin == i) | jnp.logical_not(iou <= _IOU_T)
        key_ref[...] = jnp.where(kill & has, -1.0, key)
        take = has & (cnt < _HALF_BATCH)
        acc_c = acc_c + jnp.where(take, ce_i, zero)
        if with_sl:
            acc_s = acc_s + jnp.where(take, ext(sl_ref, row, lmask), zero)
        cnt = cnt + jnp.where(has, 1, 0)
        return cnt, acc_c, acc_s

    def body(_, st):
        cnt_p, acc_pc, acc_pl, cnt_n, acc_nc = st
        cnt_p, acc_pc, acc_pl = step(keyp_ref, True, cnt_p, acc_pc, acc_pl)
        cnt_n, acc_nc, _ = step(keyn_ref, False, cnt_n, acc_nc, zero)
        return cnt_p, acc_pc, acc_pl, cnt_n, acc_nc

    cnt_p, acc_pc, acc_pl, cnt_n, acc_nc = jax.lax.fori_loop(
        0, _HALF_BATCH + 1, body,
        (jnp.int32(0), zero, zero, jnp.int32(0), zero))

    trunc_p = cnt_p > _HALF_BATCH
    trunc_n = cnt_n > _HALF_BATCH
    keep_p = jnp.minimum(cnt_p, _HALF_BATCH)
    keep_n = jnp.minimum(cnt_n, _HALF_BATCH)
    sum_pc = jnp.where(trunc_p, acc_pc, total_pc)
    sum_pl = jnp.where(trunc_p, acc_pl, total_pl)
    sum_nc = jnp.where(trunc_n, acc_nc, total_nc)
    cls_out[0, 0] = (sum_nc + sum_pc) / (keep_p + keep_n).astype(jnp.float32)
    loc_out[0, 0] = sum_pl / keep_p.astype(jnp.float32)


def kernel(cls_pred, cls_target, loc_pred, loc_target, anchors):
    cp = cls_pred[0]
    ct = cls_target[0, 0].astype(jnp.int32)
    lp = loc_pred[0]
    lt = loc_target[0]
    an = anchors[0]
    pad = _RP - _R

    def p2(x, v=0):
        return jnp.pad(x, (0, pad), constant_values=v).reshape(_ROWS, _LANES)

    args = (p2(cp[:, 0]), p2(cp[:, 1]), p2(ct, 2),
            p2(lp[:, 0]), p2(lp[:, 1]), p2(lt[:, 0]), p2(lt[:, 1]),
            p2(an[:, 0]), p2(an[:, 1]), p2(an[:, 2]), p2(an[:, 3]))
    cls_o, loc_o = pl.pallas_call(
        _ohem_kernel,
        out_shape=(jax.ShapeDtypeStruct((1, 1), jnp.float32),
                   jax.ShapeDtypeStruct((1, 1), jnp.float32)),
        out_specs=(pl.BlockSpec(memory_space=pltpu.SMEM),
                   pl.BlockSpec(memory_space=pltpu.SMEM)),
        scratch_shapes=[pltpu.VMEM((_ROWS, _LANES), jnp.float32)
                        for _ in range(5)],
    )(*args)
    return cls_o[0, 0], loc_o[0, 0]
